# baseline (speedup 1.0000x reference)
import functools

import jax
import jax.numpy as jnp
from jax.experimental import pallas as pl
from jax.experimental.pallas import tpu as pltpu

_TILE = 64


def _fused_kernel(xf_ref, rw_ref, w1_ref, w2_ref, w3_ref, out_ref,
                  e_s, r_s, s_s):
    f32 = jnp.float32
    e = pl.program_id(0)
    T = xf_ref.shape[0]
    E = rw_ref.shape[0]
    dotg = functools.partial(jax.lax.dot_general, preferred_element_type=f32)

    @pl.when(e == 0)
    def _():
        out_ref[...] = jnp.zeros_like(out_ref)
        xf = xf_ref[...]
        rw = rw_ref[...]
        logits = dotg(xf, rw, (((1,), (1,)), ((), ())))
        lmax = jnp.max(logits, axis=1, keepdims=True)
        ex = jnp.exp(logits - lmax)
        p = ex / jnp.sum(ex, axis=1, keepdims=True)

        lane = jax.lax.broadcasted_iota(jnp.int32, (T, E), 1)
        m1 = jnp.max(p, axis=1, keepdims=True)
        i1 = jnp.min(jnp.where(p == m1, lane, E), axis=1, keepdims=True)
        o1 = (lane == i1)
        pm = jnp.where(o1, -jnp.inf, p)
        m2 = jnp.max(pm, axis=1, keepdims=True)
        i2 = jnp.min(jnp.where(pm == m2, lane, E), axis=1, keepdims=True)
        o2 = (lane == i2)
        o1f = o1.astype(f32)
        o2f = o2.astype(f32)

        ssum = m1 + m2

        ones_t = jnp.ones((T, 1), f32)
        cnt1_c = dotg(o1f, ones_t, (((0,), (0,)), ((), ())))
        tr = jax.lax.broadcasted_iota(jnp.int32, (T, T), 0)
        tc = jax.lax.broadcasted_iota(jnp.int32, (T, T), 1)
        ls_t = (tc < tr).astype(f32)
        c1 = dotg(ls_t, o1f, (((1,), (0,)), ((), ())))
        rank0 = jnp.sum(o1f * c1, axis=1, keepdims=True)
        c2 = dotg(ls_t, o2f, (((1,), (0,)), ((), ())))
        rank1 = (jnp.sum(o2f * c2, axis=1, keepdims=True)
                 + dotg(o2f, cnt1_c, (((1,), (0,)), ((), ()))))

        e_s[...] = jnp.concatenate([i1, i2], axis=1)
        r_s[...] = jnp.concatenate(
            [rank0.astype(jnp.int32), rank1.astype(jnp.int32)], axis=1)
        s_s[...] = jnp.concatenate([m1 / ssum, m2 / ssum], axis=1)

    e0 = e_s[:, 0:1]
    e1 = e_s[:, 1:2]
    is0 = (e0 == e)
    is1 = (e1 == e)
    cnt = jnp.sum(is0.astype(jnp.int32) + is1.astype(jnp.int32))
    n_t = (cnt + (_TILE - 1)) // _TILE

    r0 = r_s[:, 0:1]
    r1 = r_s[:, 1:2]
    s0 = s_s[:, 0:1]
    s1 = s_s[:, 1:2]
    w1 = w1_ref[0]
    w3 = w3_ref[0]
    w2 = w2_ref[0]

    def tile_body(c, carry):
        rows = c * _TILE + jax.lax.broadcasted_iota(jnp.int32, (T, _TILE), 1)
        m0 = is0 & (r0 == rows)
        m1_ = is1 & (r1 == rows)
        g = m0.astype(f32) + m1_.astype(f32)
        gs = m0.astype(f32) * s0 + m1_.astype(f32) * s1
        xg = dotg(g, xf_ref[...], (((0,), (0,)), ((), ())))
        h1 = dotg(xg, w1, (((1,), (1,)), ((), ())))
        h3 = dotg(xg, w3, (((1,), (1,)), ((), ())))
        h = jax.nn.silu(h1) * h3
        y = dotg(h, w2, (((1,), (1,)), ((), ())))
        out_ref[...] += dotg(gs, y, (((1,), (0,)), ((), ())))
        return carry

    jax.lax.fori_loop(0, n_t, tile_body, 0)


def kernel(x, router_w, w1, w2, w3):
    orig_shape = x.shape
    H = x.shape[-1]
    xf = x.reshape(-1, H)
    T = xf.shape[0]
    E = router_w.shape[0]
    EXP = w1.shape[1]

    out = pl.pallas_call(
        _fused_kernel,
        grid=(E,),
        in_specs=[
            pl.BlockSpec((T, H), lambda e: (0, 0)),
            pl.BlockSpec((E, H), lambda e: (0, 0)),
            pl.BlockSpec((1, EXP, H), lambda e: (e, 0, 0)),
            pl.BlockSpec((1, H, EXP), lambda e: (e, 0, 0)),
            pl.BlockSpec((1, EXP, H), lambda e: (e, 0, 0)),
        ],
        out_specs=pl.BlockSpec((T, H), lambda e: (0, 0)),
        out_shape=jax.ShapeDtypeStruct((T, H), jnp.float32),
        scratch_shapes=[
            pltpu.VMEM((T, 2), jnp.int32),
            pltpu.VMEM((T, 2), jnp.int32),
            pltpu.VMEM((T, 2), jnp.float32),
        ],
        compiler_params=pltpu.CompilerParams(
            dimension_semantics=("arbitrary",)),
    )(xf, router_w, w1, w2, w3)

    return out.reshape(orig_shape)
